# baseline (speedup 1.0000x reference)
import jax
import jax.numpy as jnp
from jax.experimental import pallas as pl
from jax.experimental.pallas import tpu as pltpu

_ROWS = 2048
_COLS = 15625
_CHUNKS = 16
_CHUNK_ROWS = _ROWS // _CHUNKS


def _copy_kernel(src_ref, dst_ref, sems):
    copies = []
    for i in range(_CHUNKS):
        sl = pl.ds(i * _CHUNK_ROWS, _CHUNK_ROWS)
        c = pltpu.make_async_copy(src_ref.at[sl], dst_ref.at[sl], sems.at[i])
        c.start()
        copies.append(c)
    for c in copies:
        c.wait()


def kernel(features, labels, centers):
    del features, labels
    flat = centers.reshape(_ROWS, _COLS)
    out = pl.pallas_call(
        _copy_kernel,
        out_shape=jax.ShapeDtypeStruct((_ROWS, _COLS), jnp.float32),
        in_specs=[pl.BlockSpec(memory_space=pl.ANY)],
        out_specs=pl.BlockSpec(memory_space=pl.ANY),
        scratch_shapes=[pltpu.SemaphoreType.DMA((_CHUNKS,))],
    )(flat)
    return out.reshape(centers.shape)


# speedup vs baseline: 4.9024x; 4.9024x over previous
import functools

import jax
import jax.numpy as jnp
from jax import lax
from jax.experimental import pallas as pl
from jax.experimental.pallas import tpu as pltpu
from jax.experimental.pallas import tpu_sc as plsc

_TOTAL = 1000000 * 32
_NW = 32
_PER_W = _TOTAL // _NW
_CHUNK = 40000
_NCHUNK = _PER_W // _CHUNK
_NBUF = 3


def _make_copy():
    mesh = plsc.VectorSubcoreMesh(core_axis_name="c", subcore_axis_name="s")

    @functools.partial(
        pl.kernel,
        mesh=mesh,
        out_type=jax.ShapeDtypeStruct((_TOTAL,), jnp.float32),
        scratch_types=(
            [pltpu.VMEM((_CHUNK,), jnp.float32) for _ in range(_NBUF)]
            + [pltpu.SemaphoreType.DMA((_NBUF,)), pltpu.SemaphoreType.DMA((_NBUF,))]
        ),
    )
    def copy_k(src, dst, *rest):
        bufs, (in_sems, out_sems) = rest[:_NBUF], rest[_NBUF:]
        wid = lax.axis_index("s") * 2 + lax.axis_index("c")
        base = wid * _PER_W

        def in_copy(i, s):
            sl = pl.ds(base + i * _CHUNK, _CHUNK)
            return pltpu.make_async_copy(src.at[sl], bufs[s], in_sems.at[s])

        def out_copy(i, s):
            sl = pl.ds(base + i * _CHUNK, _CHUNK)
            return pltpu.make_async_copy(bufs[s], dst.at[sl], out_sems.at[s])

        for i in range(_NBUF):
            in_copy(i, i).start()
        for i in range(_NCHUNK):
            s = i % _NBUF
            in_copy(i, s).wait()
            out_copy(i, s).start()
            nxt = i + _NBUF
            if nxt < _NCHUNK:
                out_copy(i, s).wait()
                in_copy(nxt, s).start()
        for i in range(_NCHUNK - _NBUF, _NCHUNK):
            out_copy(i, i % _NBUF).wait()

    return copy_k


_copy = _make_copy()


def kernel(features, labels, centers):
    del features, labels
    return _copy(centers.reshape(_TOTAL)).reshape(centers.shape)


# speedup vs baseline: 5.6911x; 1.1609x over previous
import functools

import jax
import jax.numpy as jnp
from jax import lax
from jax.experimental import pallas as pl
from jax.experimental.pallas import tpu as pltpu
from jax.experimental.pallas import tpu_sc as plsc

_ROWS = 1000000
_COLS = 32
_NW = 32
_ROWS_W = 31248
_CHUNK_ROWS = 248
_NCHUNK = _ROWS_W // _CHUNK_ROWS
_NBUF = 3
_TAIL_BASE = _NW * _ROWS_W
_TAIL_ROWS = _ROWS - _TAIL_BASE


def _make_copy():
    mesh = plsc.VectorSubcoreMesh(core_axis_name="c", subcore_axis_name="s")

    @functools.partial(
        pl.kernel,
        mesh=mesh,
        out_type=jax.ShapeDtypeStruct((_ROWS, _COLS), jnp.float32),
        scratch_types=(
            [pltpu.VMEM((_CHUNK_ROWS, _COLS), jnp.float32) for _ in range(_NBUF)]
            + [pltpu.VMEM((_TAIL_ROWS, _COLS), jnp.float32),
               pltpu.SemaphoreType.DMA((_NBUF,)),
               pltpu.SemaphoreType.DMA((_NBUF,)),
               pltpu.SemaphoreType.DMA]
        ),
    )
    def copy_k(src, dst, *rest):
        bufs = rest[:_NBUF]
        tail_buf, in_sems, out_sems, tail_sem = rest[_NBUF:]
        wid = lax.axis_index("s") * 2 + lax.axis_index("c")
        base = pl.multiple_of(wid * _ROWS_W, 8)

        def in_copy(i, s):
            sl = pl.ds(base + i * _CHUNK_ROWS, _CHUNK_ROWS)
            return pltpu.make_async_copy(src.at[sl], bufs[s], in_sems.at[s])

        def out_copy(i, s):
            sl = pl.ds(base + i * _CHUNK_ROWS, _CHUNK_ROWS)
            return pltpu.make_async_copy(bufs[s], dst.at[sl], out_sems.at[s])

        for i in range(_NBUF):
            in_copy(i, i).start()
        for i in range(_NCHUNK):
            s = i % _NBUF
            in_copy(i, s).wait()
            out_copy(i, s).start()
            nxt = i + _NBUF
            if nxt < _NCHUNK:
                out_copy(i, s).wait()
                in_copy(nxt, s).start()
        for i in range(_NCHUNK - _NBUF, _NCHUNK):
            out_copy(i, i % _NBUF).wait()

        @pl.when(wid == _NW - 1)
        def _tail():
            sl = pl.ds(_TAIL_BASE, _TAIL_ROWS)
            cin = pltpu.make_async_copy(src.at[sl], tail_buf, tail_sem)
            cin.start()
            cin.wait()
            cout = pltpu.make_async_copy(tail_buf, dst.at[sl], tail_sem)
            cout.start()
            cout.wait()

    return copy_k


_copy = _make_copy()


def kernel(features, labels, centers):
    del features, labels
    return _copy(centers)


# speedup vs baseline: 48.4504x; 8.5134x over previous
import functools

import jax
import jax.numpy as jnp
from jax import lax
from jax.experimental import pallas as pl
from jax.experimental.pallas import tpu as pltpu
from jax.experimental.pallas import tpu_sc as plsc

_R = 32
_C = 1000000
_SEG_COLS = 124928
_TAIL_BASE = 8 * _SEG_COLS
_TAIL_COLS = _C - _TAIL_BASE
_CHUNK_COLS = 2048
_NCHUNK = _SEG_COLS // _CHUNK_COLS
_NBUF = 3


def _make_copy():
    mesh = plsc.VectorSubcoreMesh(core_axis_name="c", subcore_axis_name="s")

    @functools.partial(
        pl.kernel,
        mesh=mesh,
        out_type=jax.ShapeDtypeStruct((_R, _C), jnp.float32),
        scratch_types=(
            [pltpu.VMEM((8, _CHUNK_COLS), jnp.float32) for _ in range(_NBUF)]
            + [pltpu.VMEM((8, _TAIL_COLS), jnp.float32),
               pltpu.SemaphoreType.DMA((_NBUF,)),
               pltpu.SemaphoreType.DMA((_NBUF,)),
               pltpu.SemaphoreType.DMA]
        ),
    )
    def copy_k(src, dst, *rest):
        bufs = rest[:_NBUF]
        tail_buf, in_sems, out_sems, tail_sem = rest[_NBUF:]
        wid = lax.axis_index("s") * 2 + lax.axis_index("c")
        grp = lax.rem(wid, 4)
        seg = lax.div(wid, 4)
        row0 = pl.multiple_of(grp * 8, 8)
        col0 = pl.multiple_of(seg * _SEG_COLS, 128)
        rows = pl.ds(row0, 8)

        def in_copy(i, s):
            sl = pl.ds(col0 + i * _CHUNK_COLS, _CHUNK_COLS)
            return pltpu.make_async_copy(src.at[rows, sl], bufs[s], in_sems.at[s])

        def out_copy(i, s):
            sl = pl.ds(col0 + i * _CHUNK_COLS, _CHUNK_COLS)
            return pltpu.make_async_copy(bufs[s], dst.at[rows, sl], out_sems.at[s])

        for i in range(_NBUF):
            in_copy(i, i).start()
        for i in range(_NCHUNK):
            s = i % _NBUF
            in_copy(i, s).wait()
            out_copy(i, s).start()
            nxt = i + _NBUF
            if nxt < _NCHUNK:
                out_copy(i, s).wait()
                in_copy(nxt, s).start()
        for i in range(_NCHUNK - _NBUF, _NCHUNK):
            out_copy(i, i % _NBUF).wait()

        @pl.when(seg == 0)
        def _tail():
            sl = pl.ds(_TAIL_BASE, _TAIL_COLS)
            cin = pltpu.make_async_copy(src.at[rows, sl], tail_buf, tail_sem)
            cin.start()
            cin.wait()
            cout = pltpu.make_async_copy(tail_buf, dst.at[rows, sl], tail_sem)
            cout.start()
            cout.wait()

    return copy_k


_copy = _make_copy()


def kernel(features, labels, centers):
    del features, labels
    return _copy(centers.T).T
